# Initial kernel scaffold; baseline (speedup 1.0000x reference)
import jax
import jax.numpy as jnp
from jax.experimental import pallas as pl


def kernel(embedding, v_embedding, wids, vids, neg_vids):
    raise NotImplementedError("write your pallas kernel here")


# speedup vs baseline: 1.5993x; 1.5993x over previous
import functools

import jax
import jax.numpy as jnp
from jax import lax
from jax.experimental import pallas as pl
from jax.experimental.pallas import tpu as pltpu
from jax.experimental.pallas import tpu_sc as plsc

D = 64
B = 16384
NNEG = 5
T = 2 + NNEG

_info = plsc.get_sparse_core_info()
_NC, _NS = _info.num_cores, _info.num_subcores
NW = _NC * _NS

CHUNK = 128
ROWS_PER_W = B // NW

_mesh = plsc.VectorSubcoreMesh(core_axis_name="c", subcore_axis_name="s")


@functools.partial(
    pl.kernel,
    mesh=_mesh,
    out_type=jax.ShapeDtypeStruct((T * B, D), jnp.float32),
    scratch_types=[
        pltpu.VMEM((CHUNK,), jnp.int32),
        pltpu.VMEM((CHUNK, D), jnp.float32),
        pltpu.SemaphoreType.DMA,
    ],
    compiler_params=pltpu.CompilerParams(use_tc_tiling_on_sc=False),
)
def _sc_gather(emb_hbm, vemb_hbm, ids_hbm, out_hbm, idx_v, rows_v, sem):
    wid = lax.axis_index("s") * _NC + lax.axis_index("c")

    def chunk_body(table, off):
        pltpu.sync_copy(ids_hbm.at[pl.ds(off, CHUNK)], idx_v)
        pltpu.async_copy(table.at[idx_v], rows_v, sem).wait()
        pltpu.sync_copy(rows_v, out_hbm.at[pl.ds(off, CHUNK)])

    def wbody(c, _):
        chunk_body(emb_hbm, wid * ROWS_PER_W + c * CHUNK)
        return 0

    lax.fori_loop(0, ROWS_PER_W // CHUNK, wbody, 0)

    def vbody(i, _):
        t = i // (ROWS_PER_W // CHUNK) + 1
        c = i % (ROWS_PER_W // CHUNK)
        chunk_body(vemb_hbm, t * B + wid * ROWS_PER_W + c * CHUNK)
        return 0

    lax.fori_loop(0, (T - 1) * (ROWS_PER_W // CHUNK), vbody, 0)


_BB = 2048


def _score_body(rows_ref, acc_ref):
    w = rows_ref[0]
    pos = rows_ref[1]
    cs = jnp.clip(jnp.sum(w * pos, axis=1), -10.0, 10.0)
    tot = jnp.sum(jnp.log1p(jnp.exp(-cs)))
    for n in range(NNEG):
        ns = jnp.clip(jnp.sum(rows_ref[2 + n] * w, axis=1), -10.0, 10.0)
        tot += jnp.sum(jnp.log1p(jnp.exp(ns)))

    @pl.when(pl.program_id(0) == 0)
    def _():
        acc_ref[...] = jnp.zeros_like(acc_ref)

    acc_ref[...] += jnp.reshape(tot, (1, 1))


_score = pl.pallas_call(
    _score_body,
    grid=(B // _BB,),
    in_specs=[pl.BlockSpec((T, _BB, D), lambda i: (0, i, 0))],
    out_specs=pl.BlockSpec((1, 1), lambda i: (0, 0)),
    out_shape=jax.ShapeDtypeStruct((1, 1), jnp.float32),
)


def kernel(embedding, v_embedding, wids, vids, neg_vids):
    ids = jnp.concatenate([
        wids.astype(jnp.int32),
        vids.astype(jnp.int32),
        neg_vids.T.astype(jnp.int32).reshape(-1),
    ])
    rows = _sc_gather(embedding, v_embedding, ids)
    loss_sum = _score(rows.reshape(T, B, D))
    return loss_sum[0, 0] / B


# speedup vs baseline: 1.6406x; 1.0258x over previous
import functools

import jax
import jax.numpy as jnp
from jax import lax
from jax.experimental import pallas as pl
from jax.experimental.pallas import tpu as pltpu
from jax.experimental.pallas import tpu_sc as plsc

D = 64
B = 16384
NNEG = 5
T = 2 + NNEG

_info = plsc.get_sparse_core_info()
_NC, _NS = _info.num_cores, _info.num_subcores
NW = _NC * _NS

ROWS_PER_W = B // NW

_mesh = plsc.VectorSubcoreMesh(core_axis_name="c", subcore_axis_name="s")


@functools.partial(
    pl.kernel,
    mesh=_mesh,
    out_type=jax.ShapeDtypeStruct((T * B, D), jnp.float32),
    scratch_types=[
        pltpu.VMEM((T * ROWS_PER_W,), jnp.int32),
        pltpu.VMEM((ROWS_PER_W, D), jnp.float32),
        pltpu.VMEM((ROWS_PER_W, D), jnp.float32),
        pltpu.SemaphoreType.DMA,
        pltpu.SemaphoreType.DMA,
        pltpu.SemaphoreType.DMA,
        pltpu.SemaphoreType.DMA,
    ],
    compiler_params=pltpu.CompilerParams(use_tc_tiling_on_sc=False),
)
def _sc_gather(emb_hbm, vemb_hbm, ids_hbm, out_hbm,
               idx_v, buf0, buf1, g0, g1, w0, w1):
    wid = lax.axis_index("s") * _NC + lax.axis_index("c")
    bufs = (buf0, buf1)
    gsems = (g0, g1)
    wsems = (w0, w1)

    pltpu.sync_copy(ids_hbm.at[wid], idx_v)

    def table(t):
        return emb_hbm if t == 0 else vemb_hbm

    def start_gather(t):
        return pltpu.async_copy(
            table(t).at[idx_v.at[pl.ds(t * ROWS_PER_W, ROWS_PER_W)]],
            bufs[t % 2], gsems[t % 2])

    gathers = [None, None]
    writes = [None, None]
    gathers[0] = start_gather(0)
    for t in range(T):
        nxt = (t + 1) % 2
        if t + 1 < T:
            if writes[nxt] is not None:
                writes[nxt].wait()
            gathers[nxt] = start_gather(t + 1)
        gathers[t % 2].wait()
        writes[t % 2] = pltpu.async_copy(
            bufs[t % 2],
            out_hbm.at[pl.ds(t * B + wid * ROWS_PER_W, ROWS_PER_W)],
            wsems[t % 2])
    writes[0].wait()
    writes[1].wait()


_BB = 2048


def _score_body(rows_ref, acc_ref):
    w = rows_ref[0]
    pos = rows_ref[1]
    cs = jnp.clip(jnp.sum(w * pos, axis=1), -10.0, 10.0)
    tot = jnp.sum(jnp.log1p(jnp.exp(-cs)))
    for n in range(NNEG):
        ns = jnp.clip(jnp.sum(rows_ref[2 + n] * w, axis=1), -10.0, 10.0)
        tot += jnp.sum(jnp.log1p(jnp.exp(ns)))

    @pl.when(pl.program_id(0) == 0)
    def _():
        acc_ref[...] = jnp.zeros_like(acc_ref)

    acc_ref[...] += jnp.reshape(tot, (1, 1))


_score = pl.pallas_call(
    _score_body,
    grid=(B // _BB,),
    in_specs=[pl.BlockSpec((T, _BB, D), lambda i: (0, i, 0))],
    out_specs=pl.BlockSpec((1, 1), lambda i: (0, 0)),
    out_shape=jax.ShapeDtypeStruct((1, 1), jnp.float32),
)


def kernel(embedding, v_embedding, wids, vids, neg_vids):
    ids = jnp.concatenate([
        wids.astype(jnp.int32),
        vids.astype(jnp.int32),
        neg_vids.T.astype(jnp.int32).reshape(-1),
    ])
    ids = ids.reshape(T, NW, ROWS_PER_W).transpose(1, 0, 2).reshape(NW, -1)
    rows = _sc_gather(embedding, v_embedding, ids)
    loss_sum = _score(rows.reshape(T, B, D))
    return loss_sum[0, 0] / B
